# baseline (speedup 1.0000x reference)
import functools

import jax
import jax.numpy as jnp
from jax import lax
from jax.experimental import pallas as pl
from jax.experimental.pallas import tpu as pltpu
from jax.experimental.pallas import tpu_sc as plsc

_B = 16384
_C = 1000
_NW = 32
_BPW = _B // _NW
_CHUNK = 64
_NCHUNK = _BPW // _CHUNK
_L = 16
_TAB = 128

_mesh = plsc.VectorSubcoreMesh(core_axis_name="c", subcore_axis_name="s")


@functools.partial(
    pl.kernel,
    out_type=jax.ShapeDtypeStruct((_B * _C,), jnp.float32),
    mesh=_mesh,
    scratch_types=[
        pltpu.VMEM((_BPW,), jnp.int32),
        pltpu.VMEM((_TAB,), jnp.int32),
        pltpu.VMEM((_CHUNK * _C,), jnp.float32),
        pltpu.VMEM((_CHUNK * _C,), jnp.float32),
        pltpu.SemaphoreType.DMA,
        pltpu.SemaphoreType.DMA,
    ],
    compiler_params=pltpu.CompilerParams(needs_layout_passes=False),
)
def _onehot_sc(cond_hbm, tab_hbm, out_hbm, cond_v, tab_v, buf0, buf1, sem0, sem1):
    wid = lax.axis_index("s") * 2 + lax.axis_index("c")
    base = wid * _BPW

    pltpu.sync_copy(cond_hbm.at[pl.ds(base, _BPW)], cond_v)
    pltpu.sync_copy(tab_hbm, tab_v)

    zeros = jnp.zeros((_L,), jnp.float32)
    ones = jnp.ones((_L,), jnp.float32)
    iota = lax.iota(jnp.int32, _L)

    def _zero_body(i, _):
        off = i * (_L * 16)
        for k in range(16):
            buf0[pl.ds(off + k * _L, _L)] = zeros
            buf1[pl.ds(off + k * _L, _L)] = zeros
        return 0

    lax.fori_loop(0, (_CHUNK * _C) // (_L * 16), _zero_body, 0)

    bufs = (buf0, buf1)
    sems = (sem0, sem1)
    copies = [None, None]

    def _positions(k, j):
        cond16 = cond_v[pl.ds(k * _CHUNK + j * _L, _L)]
        code16 = plsc.load_gather(tab_v, [cond16])
        return (iota + j * _L) * _C + code16

    for k in range(_NCHUNK):
        b = k % 2
        buf = bufs[b]
        if k >= 2:
            copies[b].wait()
            for j in range(_CHUNK // _L):
                plsc.store_scatter(buf, [_positions(k - 2, j)], zeros)
        for j in range(_CHUNK // _L):
            plsc.store_scatter(buf, [_positions(k, j)], ones)
        dst = out_hbm.at[pl.ds((base + k * _CHUNK) * _C, _CHUNK * _C)]
        copies[b] = pltpu.async_copy(buf, dst, sems[b])

    copies[0].wait()
    copies[1].wait()


def kernel(condition, celltype_codes):
    tab = jnp.zeros((_TAB,), jnp.int32).at[:_C // 10].set(celltype_codes)
    out_flat = _onehot_sc(condition, tab)
    return out_flat.reshape(_B, _C)
